# baseline (speedup 1.0000x reference)
import functools

import jax
import jax.numpy as jnp
from jax import lax
from jax.experimental import pallas as pl
from jax.experimental.pallas import tpu as pltpu
from jax.experimental.pallas import tpu_sc as plsc

_E = 8
_TOPK = 2
_NTILES = 32

_sc_mesh = plsc.VectorSubcoreMesh(core_axis_name="c", subcore_axis_name="s")


def _make_prepare(M, K, NTOT, TPW):
    @functools.partial(
        pl.kernel, mesh=_sc_mesh,
        out_type=(jax.ShapeDtypeStruct((NTOT, K), jnp.float32),
                  jax.ShapeDtypeStruct((NTOT, 128), jnp.float32)),
        scratch_types=[
            pltpu.VMEM((TPW,), jnp.int32),
            pltpu.VMEM((TPW,), jnp.int32),
            pltpu.VMEM((TPW, K), jnp.float32),
            pltpu.VMEM((TPW, 128), jnp.float32),
            pltpu.VMEM((TPW, 128), jnp.float32),
            pltpu.SemaphoreType.DMA,
        ],
    )
    def _prepare(x_hbm, de_hbm, do_hbm, we_hbm, wo_hbm, xs_hbm, wrow_hbm,
                 idxe_v, idxo_v, rows_v, wev, wov, sem):
        wid = lax.axis_index("s") * 2 + lax.axis_index("c")
        base = wid * TPW
        pltpu.sync_copy(de_hbm.at[pl.ds(base, TPW)], idxe_v)
        pltpu.sync_copy(do_hbm.at[pl.ds(base, TPW)], idxo_v)
        pltpu.sync_copy(we_hbm.at[pl.ds(base, TPW)], wev)
        pltpu.sync_copy(wo_hbm.at[pl.ds(base, TPW)], wov)
        pltpu.sync_copy(x_hbm.at[pl.ds(base, TPW)], rows_v)
        c1 = pltpu.async_copy(rows_v, xs_hbm.at[idxe_v], sem)
        c2 = pltpu.async_copy(rows_v, xs_hbm.at[idxo_v], sem)
        c3 = pltpu.async_copy(wev, wrow_hbm.at[idxe_v], sem)
        c4 = pltpu.async_copy(wov, wrow_hbm.at[idxo_v], sem)
        c1.wait()
        c2.wait()
        c3.wait()
        c4.wait()

    return _prepare


def _make_finalize(M, K, CHT=16):
    TPT = M // _NTILES
    NCH = TPT // CHT

    @functools.partial(
        pl.kernel, mesh=_sc_mesh,
        out_type=jax.ShapeDtypeStruct((M, K), jnp.float32),
        scratch_types=[
            pltpu.VMEM((2 * CHT,), jnp.int32),
            pltpu.VMEM((2 * CHT,), jnp.int32),
            pltpu.VMEM((2 * CHT, K), jnp.float32),
            pltpu.VMEM((2 * CHT, K), jnp.float32),
            pltpu.VMEM((CHT, K), jnp.float32),
            pltpu.SemaphoreType.DMA,
            pltpu.SemaphoreType.DMA,
        ],
    )
    def _finalize(outs_hbm, gi_hbm, out_hbm, i0, i1, r0, r1, acc_v, s0, s1):
        wid = lax.axis_index("s") * 2 + lax.axis_index("c")
        tbase = wid * TPT
        idx_bufs = (i0, i1)
        row_bufs = (r0, r1)
        sems = (s0, s1)
        pltpu.sync_copy(gi_hbm.at[pl.ds(2 * tbase, 2 * CHT)], i0)
        copies = [pltpu.async_copy(outs_hbm.at[i0], r0, s0), None]
        for g in range(NCH):
            cur = g % 2
            nxt = (g + 1) % 2
            if g + 1 < NCH:
                pltpu.sync_copy(
                    gi_hbm.at[pl.ds(2 * (tbase + (g + 1) * CHT), 2 * CHT)],
                    idx_bufs[nxt])
                copies[nxt] = pltpu.async_copy(
                    outs_hbm.at[idx_bufs[nxt]], row_bufs[nxt], sems[nxt])
            copies[cur].wait()
            rb = row_bufs[cur]

            def body(j, _, rb=rb):
                for c in range(K // 16):
                    sl = pl.ds(c * 16, 16)
                    acc_v[j, sl] = rb[2 * j, sl] + rb[2 * j + 1, sl]
                return 0

            lax.fori_loop(0, CHT, body, 0)
            pltpu.sync_copy(acc_v, out_hbm.at[pl.ds(tbase + g * CHT, CHT)])

    return _finalize


def _expert_gemm_body(buf_ref, w1g_ref, w1u_ref, w2_ref, wr_ref, out_ref):
    e = pl.program_id(0)
    f = pl.program_id(1)
    ne = pl.num_programs(0)

    @pl.when(e == ne - 1)
    def _():
        out_ref[...] = jnp.zeros(out_ref.shape, out_ref.dtype)

    @pl.when(e < ne - 1)
    def _():
        xb = buf_ref[...].astype(jnp.bfloat16)
        w1g = w1g_ref[0].astype(jnp.bfloat16)
        w1u = w1u_ref[0].astype(jnp.bfloat16)
        w2b = w2_ref[0].astype(jnp.bfloat16)
        g = jax.lax.dot_general(xb, w1g, (((1,), (1,)), ((), ())),
                                preferred_element_type=jnp.float32)
        u = jax.lax.dot_general(xb, w1u, (((1,), (1,)), ((), ())),
                                preferred_element_type=jnp.float32)
        act = (g * jax.nn.sigmoid(g) * u).astype(jnp.bfloat16)
        part = jax.lax.dot_general(act, w2b, (((1,), (1,)), ((), ())),
                                   preferred_element_type=jnp.float32)
        part = part * wr_ref[:, 0:1]

        @pl.when(f == 0)
        def _():
            out_ref[...] = part

        @pl.when(f > 0)
        def _():
            out_ref[...] += part


def _expert_gemms(xs, w1, w2, wrow, C, K, dff, NTOT):
    BF = 512 if dff % 512 == 0 else dff
    NF = dff // BF
    grid = (_E + 1, NF)
    cl = lambda e: jnp.minimum(e, _E - 1)
    return pl.pallas_call(
        _expert_gemm_body,
        grid=grid,
        in_specs=[
            pl.BlockSpec((C, K), lambda e, f: (cl(e), 0)),
            pl.BlockSpec((1, BF, K), lambda e, f: (cl(e), f, 0)),
            pl.BlockSpec((1, BF, K), lambda e, f, NF=NF: (cl(e), NF + f, 0)),
            pl.BlockSpec((1, K, BF), lambda e, f: (cl(e), 0, f)),
            pl.BlockSpec((C, 128), lambda e, f: (cl(e), 0)),
        ],
        out_specs=pl.BlockSpec((C, K), lambda e, f: (e, 0)),
        out_shape=jax.ShapeDtypeStruct((NTOT, K), jnp.float32),
    )(xs, w1, w1, w2, wrow)


def kernel(x, router_logits, w1, w2):
    M, K = x.shape
    dff = w2.shape[2]
    C = (M * _TOPK // _E) * 3 // 2
    NROWS = _E * C
    NTOT = NROWS + 8

    topk_logits, topk_ids = jax.lax.top_k(router_logits, _TOPK)
    topk_weights = jax.nn.softmax(topk_logits, axis=-1)
    flat_ids = topk_ids.reshape(-1)
    one_hot = jax.nn.one_hot(flat_ids, _E, dtype=jnp.int32)
    pos = jnp.take_along_axis(jnp.cumsum(one_hot, axis=0) - 1,
                              flat_ids[:, None], axis=1)[:, 0]
    keep = pos < C
    e_rows = flat_ids * C + pos
    dest = jnp.where(keep, e_rows, NROWS).astype(jnp.int32)
    grows = jnp.where(keep, e_rows, NROWS).astype(jnp.int32)
    wflat = jnp.where(keep, topk_weights.reshape(-1), 0.0)
    de, do = dest[0::2], dest[1::2]
    we = jnp.broadcast_to(wflat[0::2, None], (M, 128))
    wo = jnp.broadcast_to(wflat[1::2, None], (M, 128))

    xs, wrow = _make_prepare(M, K, NTOT, M // _NTILES)(x, de, do, we, wo)

    out_s = _expert_gemms(xs, w1, w2, wrow, C, K, dff, NTOT)

    out = _make_finalize(M, K)(out_s, grows)
    return out


# speedup vs baseline: 1.1189x; 1.1189x over previous
import functools

import jax
import jax.numpy as jnp
from jax import lax
from jax.experimental import pallas as pl
from jax.experimental.pallas import tpu as pltpu
from jax.experimental.pallas import tpu_sc as plsc

_E = 8
_TOPK = 2
_NTILES = 32

_sc_mesh = plsc.VectorSubcoreMesh(core_axis_name="c", subcore_axis_name="s")


def _make_prepare(M, K, NTOT, TPW):
    @functools.partial(
        pl.kernel, mesh=_sc_mesh,
        out_type=jax.ShapeDtypeStruct((NTOT, K), jnp.float32),
        scratch_types=[
            pltpu.VMEM((TPW,), jnp.int32),
            pltpu.VMEM((TPW,), jnp.int32),
            pltpu.VMEM((TPW, K), jnp.float32),
            pltpu.SemaphoreType.DMA,
        ],
    )
    def _prepare(x_hbm, de_hbm, do_hbm, xs_hbm, idxe_v, idxo_v, rows_v, sem):
        wid = lax.axis_index("s") * 2 + lax.axis_index("c")
        base = wid * TPW
        pltpu.sync_copy(x_hbm.at[pl.ds(base, TPW)], rows_v)
        pltpu.sync_copy(de_hbm.at[pl.ds(base, TPW)], idxe_v)
        pltpu.sync_copy(do_hbm.at[pl.ds(base, TPW)], idxo_v)
        pltpu.async_copy(rows_v, xs_hbm.at[idxe_v], sem).wait()
        pltpu.async_copy(rows_v, xs_hbm.at[idxo_v], sem).wait()

    return _prepare


def _make_finalize(M, K, NROWS, CHT):
    TPT = M // _NTILES
    NCH = TPT // CHT

    @functools.partial(
        pl.kernel, mesh=_sc_mesh,
        out_type=jax.ShapeDtypeStruct((M, K), jnp.float32),
        scratch_types=[
            pltpu.VMEM((TPT,), jnp.int32),
            pltpu.VMEM((TPT,), jnp.int32),
            pltpu.VMEM((TPT, 16), jnp.float32),
            pltpu.VMEM((TPT, 16), jnp.float32),
            pltpu.VMEM((CHT, K), jnp.float32),
            pltpu.VMEM((CHT, K), jnp.float32),
            pltpu.VMEM((CHT, K), jnp.float32),
            pltpu.VMEM((CHT, K), jnp.float32),
            pltpu.SemaphoreType.DMA,
            pltpu.SemaphoreType.DMA,
            pltpu.SemaphoreType.DMA,
            pltpu.SemaphoreType.DMA,
        ],
    )
    def _finalize(outs_hbm, ge_hbm, go_hbm, we_hbm, wo_hbm, out_hbm,
                  ie_v, io_v, we_v, wo_v, re0, ro0, re1, ro1,
                  s0, s1, s2, s3):
        wid = lax.axis_index("s") * 2 + lax.axis_index("c")
        tbase = wid * TPT
        pltpu.sync_copy(ge_hbm.at[pl.ds(tbase, TPT)], ie_v)
        pltpu.sync_copy(go_hbm.at[pl.ds(tbase, TPT)], io_v)
        pltpu.sync_copy(we_hbm.at[pl.ds(tbase, TPT)], we_v)
        pltpu.sync_copy(wo_hbm.at[pl.ds(tbase, TPT)], wo_v)
        rbufs = ((re0, ro0), (re1, ro1))
        sems = ((s0, s1), (s2, s3))

        def start(g):
            re, ro = rbufs[g % 2]
            se, so = sems[g % 2]
            cpe = pltpu.async_copy(
                outs_hbm.at[ie_v.at[pl.ds(g * CHT, CHT)]], re, se)
            cpo = pltpu.async_copy(
                outs_hbm.at[io_v.at[pl.ds(g * CHT, CHT)]], ro, so)
            return cpe, cpo

        inflight = start(0)
        for g in range(NCH):
            nxt = start(g + 1) if g + 1 < NCH else None
            cpe, cpo = inflight
            cpe.wait()
            cpo.wait()
            re, ro = rbufs[g % 2]

            def body(j, _, re=re, ro=ro, g=g):
                wev = we_v[g * CHT + j, :]
                wov = wo_v[g * CHT + j, :]
                for c in range(K // 16):
                    sl = pl.ds(c * 16, 16)
                    re[j, sl] = wev * re[j, sl] + wov * ro[j, sl]
                return 0

            lax.fori_loop(0, CHT, body, 0)
            pltpu.sync_copy(re, out_hbm.at[pl.ds(tbase + g * CHT, CHT)])
            inflight = nxt

    return _finalize


def _expert_gemm_body(buf_ref, w1g_ref, w1u_ref, w2_ref, out_ref):
    f = pl.program_id(1)
    xb = buf_ref[...].astype(jnp.bfloat16)
    w1g = w1g_ref[0].astype(jnp.bfloat16)
    w1u = w1u_ref[0].astype(jnp.bfloat16)
    w2b = w2_ref[0].astype(jnp.bfloat16)
    g = jax.lax.dot_general(xb, w1g, (((1,), (1,)), ((), ())),
                            preferred_element_type=jnp.float32)
    u = jax.lax.dot_general(xb, w1u, (((1,), (1,)), ((), ())),
                            preferred_element_type=jnp.float32)
    act = (g * jax.nn.sigmoid(g) * u).astype(jnp.bfloat16)
    part = jax.lax.dot_general(act, w2b, (((1,), (1,)), ((), ())),
                               preferred_element_type=jnp.float32)

    @pl.when(f == 0)
    def _():
        out_ref[...] = part

    @pl.when(f > 0)
    def _():
        out_ref[...] += part


def _expert_gemms(xs, w1, w2, C, K, dff):
    BF = 512 if dff % 512 == 0 else dff
    NF = dff // BF
    grid = (_E, NF)
    return pl.pallas_call(
        _expert_gemm_body,
        grid=grid,
        in_specs=[
            pl.BlockSpec((C, K), lambda e, f: (e, 0)),
            pl.BlockSpec((1, BF, K), lambda e, f: (e, f, 0)),
            pl.BlockSpec((1, BF, K), lambda e, f, NF=NF: (e, NF + f, 0)),
            pl.BlockSpec((1, K, BF), lambda e, f: (e, 0, f)),
        ],
        out_specs=pl.BlockSpec((C, K), lambda e, f: (e, 0)),
        out_shape=jax.ShapeDtypeStruct((_E * C, K), jnp.float32),
    )(xs, w1, w1, w2)


def kernel(x, router_logits, w1, w2):
    M, K = x.shape
    dff = w2.shape[2]
    C = (M * _TOPK // _E) * 3 // 2
    NROWS = _E * C
    NTOT = NROWS + 8

    topk_logits, topk_ids = jax.lax.top_k(router_logits, _TOPK)
    topk_weights = jax.nn.softmax(topk_logits, axis=-1)
    flat_ids = topk_ids.reshape(-1)
    one_hot = jax.nn.one_hot(flat_ids, _E, dtype=jnp.int32)
    pos = jnp.take_along_axis(jnp.cumsum(one_hot, axis=0) - 1,
                              flat_ids[:, None], axis=1)[:, 0]
    keep = pos < C
    e_rows = flat_ids * C + pos
    dest = jnp.where(keep, e_rows, NROWS).astype(jnp.int32)
    grows = jnp.where(keep, e_rows, 0).astype(jnp.int32)
    wflat = jnp.where(keep, topk_weights.reshape(-1), 0.0)
    de, do = dest[0::2], dest[1::2]
    ge, go = grows[0::2], grows[1::2]
    we = jnp.broadcast_to(wflat[0::2, None], (M, 16))
    wo = jnp.broadcast_to(wflat[1::2, None], (M, 16))

    xs = _make_prepare(M, K, NTOT, M // _NTILES)(x, de, do)

    out_s = _expert_gemms(xs, w1, w2, C, K, dff)

    out = _make_finalize(M, K, NROWS, 16)(out_s, ge, go, we, wo)
    return out


# speedup vs baseline: 1.1977x; 1.0705x over previous
import functools

import jax
import jax.numpy as jnp
from jax import lax
from jax.experimental import pallas as pl
from jax.experimental.pallas import tpu as pltpu
from jax.experimental.pallas import tpu_sc as plsc

_E = 8
_TOPK = 2
_NTILES = 32

def _sc_mesh():
    return plsc.VectorSubcoreMesh(core_axis_name="c", subcore_axis_name="s")


def _make_prepare(M, K, NTOT, TPW):
    @functools.partial(
        pl.kernel, mesh=_sc_mesh(),
        out_type=jax.ShapeDtypeStruct((NTOT, K), jnp.float32),
        scratch_types=[
            pltpu.VMEM((TPW,), jnp.int32),
            pltpu.VMEM((TPW,), jnp.int32),
            pltpu.VMEM((TPW, K), jnp.float32),
            pltpu.SemaphoreType.DMA,
        ],
    )
    def _prepare(x_hbm, de_hbm, do_hbm, xs_hbm, idxe_v, idxo_v, rows_v, sem):
        wid = lax.axis_index("s") * 2 + lax.axis_index("c")
        base = wid * TPW
        pltpu.sync_copy(x_hbm.at[pl.ds(base, TPW)], rows_v)
        pltpu.sync_copy(de_hbm.at[pl.ds(base, TPW)], idxe_v)
        pltpu.sync_copy(do_hbm.at[pl.ds(base, TPW)], idxo_v)
        pltpu.async_copy(rows_v, xs_hbm.at[idxe_v], sem).wait()
        pltpu.async_copy(rows_v, xs_hbm.at[idxo_v], sem).wait()

    return _prepare


def _make_finalize(M, K, NROWS, CHT):
    TPT = M // _NTILES
    NCH = TPT // CHT

    @functools.partial(
        pl.kernel, mesh=_sc_mesh(),
        out_type=jax.ShapeDtypeStruct((M, K), jnp.float32),
        scratch_types=[
            pltpu.VMEM((TPT,), jnp.int32),
            pltpu.VMEM((TPT,), jnp.int32),
            pltpu.VMEM((TPT, 16), jnp.float32),
            pltpu.VMEM((TPT, 16), jnp.float32),
            pltpu.VMEM((CHT, K), jnp.float32),
            pltpu.VMEM((CHT, K), jnp.float32),
            pltpu.VMEM((CHT, K), jnp.float32),
            pltpu.VMEM((CHT, K), jnp.float32),
            pltpu.SemaphoreType.DMA,
            pltpu.SemaphoreType.DMA,
            pltpu.SemaphoreType.DMA,
            pltpu.SemaphoreType.DMA,
        ],
    )
    def _finalize(outs_hbm, ge_hbm, go_hbm, we_hbm, wo_hbm, out_hbm,
                  ie_v, io_v, we_v, wo_v, re0, ro0, re1, ro1,
                  s0, s1, s2, s3):
        wid = lax.axis_index("s") * 2 + lax.axis_index("c")
        tbase = wid * TPT
        pltpu.sync_copy(ge_hbm.at[pl.ds(tbase, TPT)], ie_v)
        pltpu.sync_copy(go_hbm.at[pl.ds(tbase, TPT)], io_v)
        pltpu.sync_copy(we_hbm.at[pl.ds(tbase, TPT)], we_v)
        pltpu.sync_copy(wo_hbm.at[pl.ds(tbase, TPT)], wo_v)
        rbufs = ((re0, ro0), (re1, ro1))
        sems = ((s0, s1), (s2, s3))

        def start(g):
            re, ro = rbufs[g % 2]
            se, so = sems[g % 2]
            cpe = pltpu.async_copy(
                outs_hbm.at[ie_v.at[pl.ds(g * CHT, CHT)]], re, se)
            cpo = pltpu.async_copy(
                outs_hbm.at[io_v.at[pl.ds(g * CHT, CHT)]], ro, so)
            return cpe, cpo

        inflight = start(0)
        for g in range(NCH):
            nxt = start(g + 1) if g + 1 < NCH else None
            cpe, cpo = inflight
            cpe.wait()
            cpo.wait()
            re, ro = rbufs[g % 2]

            def body(j, _, re=re, ro=ro, g=g):
                wev = we_v[g * CHT + j, :]
                wov = wo_v[g * CHT + j, :]
                for c in range(K // 16):
                    sl = pl.ds(c * 16, 16)
                    re[j, sl] = wev * re[j, sl] + wov * ro[j, sl]
                return 0

            lax.fori_loop(0, CHT, body, 0)
            pltpu.sync_copy(re, out_hbm.at[pl.ds(tbase + g * CHT, CHT)])
            inflight = nxt

    return _finalize


def _expert_gemm_body(buf_ref, w1g_ref, w1u_ref, w2_ref, out_ref):
    f = pl.program_id(1)
    xb = buf_ref[...].astype(jnp.bfloat16)
    w1g = w1g_ref[0].astype(jnp.bfloat16)
    w1u = w1u_ref[0].astype(jnp.bfloat16)
    w2b = w2_ref[0].astype(jnp.bfloat16)
    g = jax.lax.dot_general(xb, w1g, (((1,), (1,)), ((), ())),
                            preferred_element_type=jnp.float32)
    u = jax.lax.dot_general(xb, w1u, (((1,), (1,)), ((), ())),
                            preferred_element_type=jnp.float32)
    act = (g * jax.nn.sigmoid(g) * u).astype(jnp.bfloat16)
    part = jax.lax.dot_general(act, w2b, (((1,), (1,)), ((), ())),
                               preferred_element_type=jnp.float32)

    @pl.when(f == 0)
    def _():
        out_ref[...] = part

    @pl.when(f > 0)
    def _():
        out_ref[...] += part


def _expert_gemms(xs, w1, w2, C, K, dff):
    BF = 512 if dff % 512 == 0 else dff
    NF = dff // BF
    grid = (_E, NF)
    return pl.pallas_call(
        _expert_gemm_body,
        grid=grid,
        in_specs=[
            pl.BlockSpec((C, K), lambda e, f: (e, 0)),
            pl.BlockSpec((1, BF, K), lambda e, f: (e, f, 0)),
            pl.BlockSpec((1, BF, K), lambda e, f, NF=NF: (e, NF + f, 0)),
            pl.BlockSpec((1, K, BF), lambda e, f: (e, 0, f)),
        ],
        out_specs=pl.BlockSpec((C, K), lambda e, f: (e, 0)),
        out_shape=jax.ShapeDtypeStruct((_E * C, K), jnp.float32),
    )(xs, w1, w1, w2)


def _routing_body(C, NROWS, lt_ref, dests_ref, grows_ref, wts_ref):
    lt = lt_ref[...]
    E, M = lt.shape
    iota = jax.lax.broadcasted_iota(jnp.int32, (E, M), 0)
    m1 = jnp.max(lt, axis=0, keepdims=True)
    a1 = jnp.min(jnp.where(lt == m1, iota, E), axis=0, keepdims=True)
    oh0 = iota == a1
    masked = jnp.where(oh0, -jnp.inf, lt)
    m2 = jnp.max(masked, axis=0, keepdims=True)
    a2 = jnp.min(jnp.where(masked == m2, iota, E), axis=0, keepdims=True)
    oh1 = iota == a2
    w0 = jax.nn.sigmoid(m1 - m2)
    w1 = jax.nn.sigmoid(m2 - m1)
    toh = oh0.astype(jnp.float32) + oh1.astype(jnp.float32)
    r = jax.lax.broadcasted_iota(jnp.int32, (128, 128), 0)
    cc = jax.lax.broadcasted_iota(jnp.int32, (128, 128), 1)
    U = (r < cc).astype(jnp.bfloat16)
    chunks = []
    base = jnp.zeros((E, 1), jnp.float32)
    for c in range(M // 128):
        tc = toh[:, c * 128:(c + 1) * 128]
        s_in = jax.lax.dot_general(tc.astype(jnp.bfloat16), U,
                                   (((1,), (0,)), ((), ())),
                                   preferred_element_type=jnp.float32)
        chunks.append(s_in + base)
        base = base + jnp.sum(tc, axis=1, keepdims=True)
    S = jnp.concatenate(chunks, axis=1)
    pos0 = jnp.sum(jnp.where(oh0, S, 0.0), axis=0,
                   keepdims=True).astype(jnp.int32)
    pos1 = jnp.sum(jnp.where(oh1, S, 0.0), axis=0,
                   keepdims=True).astype(jnp.int32)

    def make(a, pos, w):
        keepm = pos < C
        er = a * C + pos
        return (jnp.where(keepm, er, NROWS), jnp.where(keepm, er, 0),
                jnp.where(keepm, w, 0.0))

    d0, g0, wk0 = make(a1, pos0, w0)
    d1, g1, wk1 = make(a2, pos1, w1)
    dests_ref[...] = jnp.concatenate([d0, d1], axis=0)
    grows_ref[...] = jnp.concatenate([g0, g1], axis=0)
    wts_ref[...] = jnp.concatenate([wk0, wk1], axis=0)


def _routing(logits_t, C, NROWS):
    E, M = logits_t.shape
    return pl.pallas_call(
        functools.partial(_routing_body, C, NROWS),
        out_shape=(jax.ShapeDtypeStruct((2, M), jnp.int32),
                   jax.ShapeDtypeStruct((2, M), jnp.int32),
                   jax.ShapeDtypeStruct((2, M), jnp.float32)),
    )(logits_t)


def kernel(x, router_logits, w1, w2):
    M, K = x.shape
    dff = w2.shape[2]
    C = (M * _TOPK // _E) * 3 // 2
    NROWS = _E * C
    NTOT = NROWS + 8

    dests, growsp, wts = _routing(router_logits.T, C, NROWS)
    de, do = dests[0], dests[1]
    ge, go = growsp[0], growsp[1]
    we = jnp.broadcast_to(wts[0][:, None], (M, 16))
    wo = jnp.broadcast_to(wts[1][:, None], (M, 16))

    xs = _make_prepare(M, K, NTOT, M // _NTILES)(x, de, do)

    out_s = _expert_gemms(xs, w1, w2, C, K, dff)

    out = _make_finalize(M, K, NROWS, 16)(out_s, ge, go, we, wo)
    return out


# speedup vs baseline: 1.2815x; 1.0700x over previous
import functools

import jax
import jax.numpy as jnp
from jax import lax
from jax.experimental import pallas as pl
from jax.experimental.pallas import tpu as pltpu
from jax.experimental.pallas import tpu_sc as plsc

_E = 8
_TOPK = 2
_NTILES = 32

def _sc_mesh():
    return plsc.VectorSubcoreMesh(core_axis_name="c", subcore_axis_name="s")


def _make_prepare(M, K, NTOT, TPW):
    @functools.partial(
        pl.kernel, mesh=_sc_mesh(),
        out_type=jax.ShapeDtypeStruct((NTOT, K), jnp.float32),
        scratch_types=[
            pltpu.VMEM((TPW,), jnp.int32),
            pltpu.VMEM((TPW,), jnp.int32),
            pltpu.VMEM((TPW, K), jnp.float32),
            pltpu.SemaphoreType.DMA,
        ],
    )
    def _prepare(x_hbm, de_hbm, do_hbm, xs_hbm, idxe_v, idxo_v, rows_v, sem):
        wid = lax.axis_index("s") * 2 + lax.axis_index("c")
        base = wid * TPW
        pltpu.sync_copy(x_hbm.at[pl.ds(base, TPW)], rows_v)
        pltpu.sync_copy(de_hbm.at[pl.ds(base, TPW)], idxe_v)
        pltpu.sync_copy(do_hbm.at[pl.ds(base, TPW)], idxo_v)
        pltpu.async_copy(rows_v, xs_hbm.at[idxe_v], sem).wait()
        pltpu.async_copy(rows_v, xs_hbm.at[idxo_v], sem).wait()

    return _prepare


def _make_finalize(M, K, NROWS, CHT):
    TPT = M // _NTILES
    NCH = TPT // CHT

    @functools.partial(
        pl.kernel, mesh=_sc_mesh(),
        out_type=jax.ShapeDtypeStruct((M, K), jnp.float32),
        scratch_types=[
            pltpu.VMEM((TPT,), jnp.int32),
            pltpu.VMEM((TPT,), jnp.int32),
            pltpu.VMEM((TPT, 16), jnp.float32),
            pltpu.VMEM((TPT, 16), jnp.float32),
            pltpu.VMEM((CHT, K), jnp.float32),
            pltpu.VMEM((CHT, K), jnp.float32),
            pltpu.VMEM((CHT, K), jnp.float32),
            pltpu.VMEM((CHT, K), jnp.float32),
            pltpu.SemaphoreType.DMA,
            pltpu.SemaphoreType.DMA,
            pltpu.SemaphoreType.DMA,
            pltpu.SemaphoreType.DMA,
        ],
    )
    def _finalize(outs_hbm, ge_hbm, go_hbm, we_hbm, wo_hbm, out_hbm,
                  ie_v, io_v, we_v, wo_v, re0, ro0, re1, ro1,
                  s0, s1, s2, s3):
        wid = lax.axis_index("s") * 2 + lax.axis_index("c")
        tbase = wid * TPT
        pltpu.sync_copy(ge_hbm.at[pl.ds(tbase, TPT)], ie_v)
        pltpu.sync_copy(go_hbm.at[pl.ds(tbase, TPT)], io_v)
        pltpu.sync_copy(we_hbm.at[pl.ds(tbase, TPT)], we_v)
        pltpu.sync_copy(wo_hbm.at[pl.ds(tbase, TPT)], wo_v)
        rbufs = ((re0, ro0), (re1, ro1))
        sems = ((s0, s1), (s2, s3))

        def start(g):
            re, ro = rbufs[g % 2]
            se, so = sems[g % 2]
            cpe = pltpu.async_copy(
                outs_hbm.at[ie_v.at[pl.ds(g * CHT, CHT)]], re, se)
            cpo = pltpu.async_copy(
                outs_hbm.at[io_v.at[pl.ds(g * CHT, CHT)]], ro, so)
            return cpe, cpo

        inflight = start(0)
        for g in range(NCH):
            nxt = start(g + 1) if g + 1 < NCH else None
            cpe, cpo = inflight
            cpe.wait()
            cpo.wait()
            re, ro = rbufs[g % 2]

            def body(j, _, re=re, ro=ro, g=g):
                wev = we_v[g * CHT + j, :]
                wov = wo_v[g * CHT + j, :]
                for c in range(K // 16):
                    sl = pl.ds(c * 16, 16)
                    re[j, sl] = wev * re[j, sl] + wov * ro[j, sl]
                return 0

            lax.fori_loop(0, CHT, body, 0)
            pltpu.sync_copy(re, out_hbm.at[pl.ds(tbase + g * CHT, CHT)])
            inflight = nxt

    return _finalize


def _expert_gemm_body(buf_ref, w1g_ref, w1u_ref, w2_ref, out_ref):
    f = pl.program_id(1)
    xb = buf_ref[...].astype(jnp.bfloat16)
    w1g = w1g_ref[0].astype(jnp.bfloat16)
    w1u = w1u_ref[0].astype(jnp.bfloat16)
    w2b = w2_ref[0].astype(jnp.bfloat16)
    g = jax.lax.dot_general(xb, w1g, (((1,), (1,)), ((), ())),
                            preferred_element_type=jnp.float32)
    u = jax.lax.dot_general(xb, w1u, (((1,), (1,)), ((), ())),
                            preferred_element_type=jnp.float32)
    act = (g * jax.nn.sigmoid(g) * u).astype(jnp.bfloat16)
    part = jax.lax.dot_general(act, w2b, (((1,), (1,)), ((), ())),
                               preferred_element_type=jnp.float32)

    @pl.when(f == 0)
    def _():
        out_ref[...] = part

    @pl.when(f > 0)
    def _():
        out_ref[...] += part


def _expert_gemms(xs, w1, w2, C, K, dff):
    BF = 1024 if dff % 1024 == 0 else dff
    NF = dff // BF
    grid = (_E, NF)
    return pl.pallas_call(
        _expert_gemm_body,
        grid=grid,
        in_specs=[
            pl.BlockSpec((C, K), lambda e, f: (e, 0)),
            pl.BlockSpec((1, BF, K), lambda e, f: (e, f, 0)),
            pl.BlockSpec((1, BF, K), lambda e, f, NF=NF: (e, NF + f, 0)),
            pl.BlockSpec((1, K, BF), lambda e, f: (e, 0, f)),
        ],
        out_specs=pl.BlockSpec((C, K), lambda e, f: (e, 0)),
        out_shape=jax.ShapeDtypeStruct((_E * C, K), jnp.float32),
    )(xs, w1, w1, w2)


def _routing_body(C, NROWS, lt_ref, dests_ref, grows_ref, wts_ref):
    lt = lt_ref[...]
    E, M = lt.shape
    iota = jax.lax.broadcasted_iota(jnp.int32, (E, M), 0)
    m1 = jnp.max(lt, axis=0, keepdims=True)
    a1 = jnp.min(jnp.where(lt == m1, iota, E), axis=0, keepdims=True)
    oh0 = iota == a1
    masked = jnp.where(oh0, -jnp.inf, lt)
    m2 = jnp.max(masked, axis=0, keepdims=True)
    a2 = jnp.min(jnp.where(masked == m2, iota, E), axis=0, keepdims=True)
    oh1 = iota == a2
    w0 = jax.nn.sigmoid(m1 - m2)
    w1 = jax.nn.sigmoid(m2 - m1)
    toh = oh0.astype(jnp.float32) + oh1.astype(jnp.float32)
    r = jax.lax.broadcasted_iota(jnp.int32, (128, 128), 0)
    cc = jax.lax.broadcasted_iota(jnp.int32, (128, 128), 1)
    U = (r < cc).astype(jnp.bfloat16)
    chunks = []
    base = jnp.zeros((E, 1), jnp.float32)
    for c in range(M // 128):
        tc = toh[:, c * 128:(c + 1) * 128]
        s_in = jax.lax.dot_general(tc.astype(jnp.bfloat16), U,
                                   (((1,), (0,)), ((), ())),
                                   preferred_element_type=jnp.float32)
        chunks.append(s_in + base)
        base = base + jnp.sum(tc, axis=1, keepdims=True)
    S = jnp.concatenate(chunks, axis=1)
    pos0 = jnp.sum(jnp.where(oh0, S, 0.0), axis=0,
                   keepdims=True).astype(jnp.int32)
    pos1 = jnp.sum(jnp.where(oh1, S, 0.0), axis=0,
                   keepdims=True).astype(jnp.int32)

    def make(a, pos, w):
        keepm = pos < C
        er = a * C + pos
        return (jnp.where(keepm, er, NROWS), jnp.where(keepm, er, 0),
                jnp.where(keepm, w, 0.0))

    d0, g0, wk0 = make(a1, pos0, w0)
    d1, g1, wk1 = make(a2, pos1, w1)
    dests_ref[...] = jnp.concatenate([d0, d1], axis=0)
    grows_ref[...] = jnp.concatenate([g0, g1], axis=0)
    wts_ref[...] = jnp.concatenate([wk0, wk1], axis=0)


def _routing(logits_t, C, NROWS):
    E, M = logits_t.shape
    return pl.pallas_call(
        functools.partial(_routing_body, C, NROWS),
        out_shape=(jax.ShapeDtypeStruct((2, M), jnp.int32),
                   jax.ShapeDtypeStruct((2, M), jnp.int32),
                   jax.ShapeDtypeStruct((2, M), jnp.float32)),
    )(logits_t)


def kernel(x, router_logits, w1, w2):
    M, K = x.shape
    dff = w2.shape[2]
    C = (M * _TOPK // _E) * 3 // 2
    NROWS = _E * C
    NTOT = NROWS + 8

    dests, growsp, wts = _routing(router_logits.T, C, NROWS)
    de, do = dests[0], dests[1]
    ge, go = growsp[0], growsp[1]
    we = jnp.broadcast_to(wts[0][:, None], (M, 16))
    wo = jnp.broadcast_to(wts[1][:, None], (M, 16))

    xs = _make_prepare(M, K, NTOT, M // _NTILES)(x, de, do)

    out_s = _expert_gemms(xs, w1, w2, C, K, dff)

    out = _make_finalize(M, K, NROWS, 16)(out_s, ge, go, we, wo)
    return out
